# baseline (speedup 1.0000x reference)
import functools

import jax
import jax.numpy as jnp
from jax import lax
from jax.experimental import pallas as pl
from jax.experimental.pallas import tpu as pltpu
from jax.experimental.pallas import tpu_sc as plsc

_E = 1600000
_NODES = 50000
_CELLS = 100000
_CELLSP = 100096
_NGC = _CELLSP // 128
_NC, _NS = 2, 16
_NW = _NC * _NS
_EC = 1024
_NCH = _E // _EC
_TAIL_E = _E - _NCH * _EC
_NPS = _NODES // _NS
_ZR = 625
_WB = 3120


def _scatter_body(ea, nei, out0, out1, rbuf, sbufA, sbufB, ibuf, zbuf, table):
    c = lax.axis_index("c")
    s = lax.axis_index("s")
    w = s * _NC + c

    def _zb(i, carry):
        zbuf[i] = jnp.zeros((16,), jnp.float32)
        return carry

    lax.fori_loop(0, _ZR, _zb, 0)
    for k in range(_NPS // _ZR):
        pltpu.sync_copy(zbuf, table.at[pl.ds(s * _NPS + k * _ZR, _ZR)])
    plsc.subcore_barrier()

    cs = w * _NCH // _NW
    ce = (w + 1) * _NCH // _NW
    iota = lax.iota(jnp.int32, 16)
    iota_b = iota + 16

    def _repack(n_edges):
        def _rp(i4, carry):
            for k in range(4):
                i = i4 * 4 + k
                col = jnp.full((16,), 0, jnp.int32) + i
                sbufA[i] = plsc.load_gather(rbuf, [iota, col])
                sbufB[i] = plsc.load_gather(rbuf, [iota_b, col])
            return carry

        lax.fori_loop(0, n_edges // 4, _rp, 0)

    def _chunk(ci, carry):
        pltpu.sync_copy(ea.at[:, pl.ds(ci * _EC, _EC)], rbuf)
        pltpu.sync_copy(nei.at[0, pl.ds(ci * 8, 8)], ibuf.at[0])
        pltpu.sync_copy(nei.at[1, pl.ds(ci * 8, 8)], ibuf.at[1])
        _repack(_EC)
        for j in range(8):
            pltpu.sync_copy(sbufA.at[pl.ds(j * 128, 128)],
                            table.at[ibuf.at[0, j]], add=True)
            pltpu.sync_copy(sbufB.at[pl.ds(j * 128, 128)],
                            table.at[ibuf.at[1, j]], add=True)
        return carry

    lax.fori_loop(cs, ce, _chunk, 0)

    @pl.when(w == 0)
    def _():
        pltpu.sync_copy(ea.at[:, pl.ds(_NCH * _EC, _TAIL_E)],
                        rbuf.at[:, pl.ds(0, _TAIL_E)])
        pltpu.sync_copy(nei.at[0, pl.ds(_NCH * 8, 4)], ibuf.at[0, pl.ds(0, 4)])
        pltpu.sync_copy(nei.at[1, pl.ds(_NCH * 8, 4)], ibuf.at[1, pl.ds(0, 4)])
        _repack(_TAIL_E)
        for j in range(4):
            pltpu.sync_copy(sbufA.at[pl.ds(j * 128, 128)],
                            table.at[ibuf.at[0, j]], add=True)
            pltpu.sync_copy(sbufB.at[pl.ds(j * 128, 128)],
                            table.at[ibuf.at[1, j]], add=True)

    plsc.subcore_barrier()

    @pl.when(c == 0)
    def _():
        pltpu.sync_copy(table.at[pl.ds(s * _WB, _WB)],
                        out0.at[pl.ds(s * _WB, _WB)])

        @pl.when(s == _NS - 1)
        def _():
            pltpu.sync_copy(table.at[pl.ds(_NS * _WB, _NODES - _NS * _WB)],
                            out0.at[pl.ds(_NS * _WB, _NODES - _NS * _WB)])

    @pl.when(c == 1)
    def _():
        pltpu.sync_copy(table.at[pl.ds(s * _WB, _WB)],
                        out1.at[pl.ds(s * _WB, _WB)])

        @pl.when(s == _NS - 1)
        def _():
            pltpu.sync_copy(table.at[pl.ds(_NS * _WB, _NODES - _NS * _WB)],
                            out1.at[pl.ds(_NS * _WB, _NODES - _NS * _WB)])


def _gather_body(p0, p1, fc, cells, ibuf, rbuf, obuf, sem):
    c = lax.axis_index("c")
    s = lax.axis_index("s")
    w = s * _NC + c
    gs = w * _NGC // _NW
    ge = (w + 1) * _NGC // _NW

    def _grp(g, carry):
        for j in range(3):
            pltpu.sync_copy(fc.at[j, g], ibuf.at[j])
        cps = []
        for j in range(3):
            cps.append(pltpu.async_copy(p0.at[ibuf.at[j]], rbuf.at[j], sem))
            cps.append(pltpu.async_copy(p1.at[ibuf.at[j]], rbuf.at[3 + j], sem))
        for cp in cps:
            cp.wait()

        def _cell(i, cc):
            acc = ((rbuf[0, i] + rbuf[1, i]) + (rbuf[2, i] + rbuf[3, i])
                   + (rbuf[4, i] + rbuf[5, i]))
            obuf[i] = acc
            return cc

        lax.fori_loop(0, 128, _cell, 0)
        pltpu.sync_copy(obuf, cells.at[pl.ds(g * 128, 128)])
        return carry

    lax.fori_loop(gs, ge, _grp, 0)


def _mm_body(x_ref, cl_ref, wx_ref, wc_ref, b_ref, o_ref):
    o_ref[...] = (jnp.dot(x_ref[...], wx_ref[...],
                          preferred_element_type=jnp.float32)
                  + jnp.dot(cl_ref[...], wc_ref[...],
                            preferred_element_type=jnp.float32)
                  + b_ref[...])


def _make_sc_kernels():
    mesh = plsc.VectorSubcoreMesh(core_axis_name="c", subcore_axis_name="s",
                                  num_cores=_NC, num_subcores=_NS)
    scatter = pl.kernel(
        _scatter_body,
        compiler_params=pltpu.CompilerParams(use_tc_tiling_on_sc=False,
                                             needs_layout_passes=False),
        out_type=(jax.ShapeDtypeStruct((_NODES, 16), jnp.float32),
                  jax.ShapeDtypeStruct((_NODES, 16), jnp.float32)),
        mesh=mesh,
        scratch_types=[
            pltpu.VMEM((32, _EC), jnp.float32),
            pltpu.VMEM((_EC, 16), jnp.float32),
            pltpu.VMEM((_EC, 16), jnp.float32),
            pltpu.VMEM((2, 8, 128), jnp.int32),
            pltpu.VMEM((_ZR, 16), jnp.float32),
            pltpu.VMEM_SHARED((_NODES, 16), jnp.float32),
        ],
    )
    gather = pl.kernel(
        _gather_body,
        compiler_params=pltpu.CompilerParams(use_tc_tiling_on_sc=False),
        out_type=jax.ShapeDtypeStruct((_CELLSP, 16), jnp.float32),
        mesh=mesh,
        scratch_types=[
            pltpu.VMEM((3, 128), jnp.int32),
            pltpu.VMEM((6, 128, 16), jnp.float32),
            pltpu.VMEM((128, 16), jnp.float32),
            pltpu.SemaphoreType.DMA,
        ],
    )
    return scatter, gather


def _matmul(x, cl, wx, wc, b2):
    blk = 2000
    return pl.pallas_call(
        _mm_body,
        grid=(_CELLS // blk,),
        in_specs=[
            pl.BlockSpec((blk, 128), lambda i: (i, 0)),
            pl.BlockSpec((blk, 16), lambda i: (i, 0)),
            pl.BlockSpec((128, 128), lambda i: (0, 0)),
            pl.BlockSpec((16, 128), lambda i: (0, 0)),
            pl.BlockSpec((1, 128), lambda i: (0, 0)),
        ],
        out_specs=pl.BlockSpec((blk, 128), lambda i: (i, 0)),
        out_shape=jax.ShapeDtypeStruct((_CELLS, 128), jnp.float32),
    )(x, cl, wx, wc, b2)


def kernel(x, edge_attr, edge_index, node_edge_index, face, W, b):
    nei_i = node_edge_index.astype(jnp.int32).reshape(2, _E // 128, 128)
    fcp = jnp.pad(face.astype(jnp.int32),
                  ((0, 0), (0, _CELLSP - _CELLS))).reshape(3, _NGC, 128)
    scatter, gather = _make_sc_kernels()
    p0, p1 = scatter(edge_attr.T, nei_i)
    cells = gather(p0, p1, fcp)
    wx = W[:128]
    wc = W[128:] * (1.0 / 3.0)
    out = _matmul(x, cells, wx, wc, b.reshape(1, 128))
    return (out, edge_attr, edge_index)


# speedup vs baseline: 4.7851x; 4.7851x over previous
import functools

import jax
import jax.numpy as jnp
from jax import lax
from jax.experimental import pallas as pl
from jax.experimental.pallas import tpu as pltpu
from jax.experimental.pallas import tpu_sc as plsc

_E = 1600000
_NGE = _E // 128
_NODES = 50000
_CELLS = 100000
_CELLSP = 100096
_NGC = _CELLSP // 128
_NC, _NS = 2, 16
_NW = _NC * _NS
_CH = 8
_EC = _CH * 128
_NCHK = _NGE // _CH
_NPS = _NODES // _NS
_ZR = 625


def _scatter_body(ea, nei, out0, out1, rbufA, rbufB, ibuf, zbuf, table,
                  lsem0, lsem1, ssem):
    c = lax.axis_index("c")
    s = lax.axis_index("s")
    w = s * _NC + c

    def _zb(i, carry):
        zbuf[i] = jnp.zeros((16,), jnp.float32)
        return carry

    lax.fori_loop(0, _ZR, _zb, 0)
    for k in range(_NPS // _ZR):
        pltpu.sync_copy(zbuf, table.at[pl.ds(s * _NPS + k * _ZR, _ZR)])
    plsc.subcore_barrier()

    cs = w * _NCHK // _NW
    ce = (w + 1) * _NCHK // _NW

    def _loads(ci, p, sem, start):
        g0 = ci * _CH
        e0 = g0 * 128
        pairs = (
            (ea.at[pl.ds(e0, _EC), pl.ds(0, 16)], rbufA.at[p]),
            (ea.at[pl.ds(e0, _EC), pl.ds(16, 16)], rbufB.at[p]),
            (nei.at[0, pl.ds(g0, _CH)], ibuf.at[0, p]),
            (nei.at[1, pl.ds(g0, _CH)], ibuf.at[1, p]),
        )
        for src, dst in pairs:
            if start:
                pltpu.async_copy(src, dst, sem)
            else:
                pltpu.make_async_copy(src, dst, sem).wait()

    def _chunk(ci, carry):
        p = lax.rem(ci - cs, 2)

        @pl.when(p == 0)
        def _():
            _loads(ci, 0, lsem0, start=False)

        @pl.when(p == 1)
        def _():
            _loads(ci, 1, lsem1, start=False)

        @pl.when((p == 0) & (ci + 1 < ce))
        def _():
            _loads(ci + 1, 1, lsem1, start=True)

        @pl.when((p == 1) & (ci + 1 < ce))
        def _():
            _loads(ci + 1, 0, lsem0, start=True)

        cps = []
        for j in range(_CH):
            cps.append(pltpu.async_copy(rbufA.at[p, pl.ds(j * 128, 128)],
                                        table.at[ibuf.at[0, p, j]], ssem,
                                        add=True))
            cps.append(pltpu.async_copy(rbufB.at[p, pl.ds(j * 128, 128)],
                                        table.at[ibuf.at[1, p, j]], ssem,
                                        add=True))
        for cp in cps:
            cp.wait()
        return carry

    @pl.when(cs < ce)
    def _():
        _loads(cs, 0, lsem0, start=True)

    lax.fori_loop(cs, ce, _chunk, 0)

    @pl.when(w == 0)
    def _():
        e0 = _NCHK * _EC
        g0 = _NCHK * _CH
        pltpu.sync_copy(ea.at[pl.ds(e0, 512), pl.ds(0, 16)],
                        rbufA.at[0, pl.ds(0, 512)])
        pltpu.sync_copy(ea.at[pl.ds(e0, 512), pl.ds(16, 16)],
                        rbufB.at[0, pl.ds(0, 512)])
        pltpu.sync_copy(nei.at[0, pl.ds(g0, 4)], ibuf.at[0, 0, pl.ds(0, 4)])
        pltpu.sync_copy(nei.at[1, pl.ds(g0, 4)], ibuf.at[1, 0, pl.ds(0, 4)])
        for j in range(4):
            pltpu.sync_copy(rbufA.at[0, pl.ds(j * 128, 128)],
                            table.at[ibuf.at[0, 0, j]], add=True)
            pltpu.sync_copy(rbufB.at[0, pl.ds(j * 128, 128)],
                            table.at[ibuf.at[1, 0, j]], add=True)

    plsc.subcore_barrier()

    @pl.when(c == 0)
    def _():
        pltpu.sync_copy(table.at[pl.ds(s * _NPS, _NPS)],
                        out0.at[pl.ds(s * _NPS, _NPS)])

    @pl.when(c == 1)
    def _():
        pltpu.sync_copy(table.at[pl.ds(s * _NPS, _NPS)],
                        out1.at[pl.ds(s * _NPS, _NPS)])


def _gather_body(p0, p1, fc, cells, ibuf, rbuf, obuf, sem):
    c = lax.axis_index("c")
    s = lax.axis_index("s")
    w = s * _NC + c
    gs = w * _NGC // _NW
    ge = (w + 1) * _NGC // _NW

    def _grp(g, carry):
        for j in range(3):
            pltpu.sync_copy(fc.at[j, g], ibuf.at[j])
        cps = []
        for j in range(3):
            cps.append(pltpu.async_copy(p0.at[ibuf.at[j]], rbuf.at[j], sem))
            cps.append(pltpu.async_copy(p1.at[ibuf.at[j]], rbuf.at[3 + j], sem))
        for cp in cps:
            cp.wait()

        def _cell(i, cc):
            acc = ((rbuf[0, i] + rbuf[1, i]) + (rbuf[2, i] + rbuf[3, i])
                   + (rbuf[4, i] + rbuf[5, i]))
            obuf[i] = acc
            return cc

        lax.fori_loop(0, 128, _cell, 0)
        pltpu.sync_copy(obuf, cells.at[pl.ds(g * 128, 128)])
        return carry

    lax.fori_loop(gs, ge, _grp, 0)


def _mm_body(x_ref, cl_ref, wx_ref, wc_ref, b_ref, o_ref):
    o_ref[...] = (jnp.dot(x_ref[...], wx_ref[...],
                          preferred_element_type=jnp.float32)
                  + jnp.dot(cl_ref[...], wc_ref[...],
                            preferred_element_type=jnp.float32)
                  + b_ref[...])


def _make_sc_kernels():
    mesh = plsc.VectorSubcoreMesh(core_axis_name="c", subcore_axis_name="s",
                                  num_cores=_NC, num_subcores=_NS)
    params = pltpu.CompilerParams(use_tc_tiling_on_sc=False)
    scatter = pl.kernel(
        _scatter_body,
        compiler_params=params,
        out_type=(jax.ShapeDtypeStruct((_NODES, 16), jnp.float32),
                  jax.ShapeDtypeStruct((_NODES, 16), jnp.float32)),
        mesh=mesh,
        scratch_types=[
            pltpu.VMEM((2, _EC, 16), jnp.float32),
            pltpu.VMEM((2, _EC, 16), jnp.float32),
            pltpu.VMEM((2, 2, _CH, 128), jnp.int32),
            pltpu.VMEM((_ZR, 16), jnp.float32),
            pltpu.VMEM_SHARED((_NODES, 16), jnp.float32),
            pltpu.SemaphoreType.DMA,
            pltpu.SemaphoreType.DMA,
            pltpu.SemaphoreType.DMA,
        ],
    )
    gather = pl.kernel(
        _gather_body,
        compiler_params=params,
        out_type=jax.ShapeDtypeStruct((_CELLSP, 16), jnp.float32),
        mesh=mesh,
        scratch_types=[
            pltpu.VMEM((3, 128), jnp.int32),
            pltpu.VMEM((6, 128, 16), jnp.float32),
            pltpu.VMEM((128, 16), jnp.float32),
            pltpu.SemaphoreType.DMA,
        ],
    )
    return scatter, gather


def _matmul(x, cl, wx, wc, b2):
    blk = 2000
    return pl.pallas_call(
        _mm_body,
        grid=(_CELLS // blk,),
        in_specs=[
            pl.BlockSpec((blk, 128), lambda i: (i, 0)),
            pl.BlockSpec((blk, 16), lambda i: (i, 0)),
            pl.BlockSpec((128, 128), lambda i: (0, 0)),
            pl.BlockSpec((16, 128), lambda i: (0, 0)),
            pl.BlockSpec((1, 128), lambda i: (0, 0)),
        ],
        out_specs=pl.BlockSpec((blk, 128), lambda i: (i, 0)),
        out_shape=jax.ShapeDtypeStruct((_CELLS, 128), jnp.float32),
    )(x, cl, wx, wc, b2)


def kernel(x, edge_attr, edge_index, node_edge_index, face, W, b):
    nei = node_edge_index.astype(jnp.int32).reshape(2, _NGE, 128)
    fcp = jnp.pad(face.astype(jnp.int32),
                  ((0, 0), (0, _CELLSP - _CELLS))).reshape(3, _NGC, 128)
    scatter, gather = _make_sc_kernels()
    p0, p1 = scatter(edge_attr, nei)
    cells = gather(p0, p1, fcp)
    wx = W[:128]
    wc = W[128:] * (1.0 / 3.0)
    out = _matmul(x, cells, wx, wc, b.reshape(1, 128))
    return (out, edge_attr, edge_index)


# speedup vs baseline: 4.8568x; 1.0150x over previous
import functools

import jax
import jax.numpy as jnp
from jax import lax
from jax.experimental import pallas as pl
from jax.experimental.pallas import tpu as pltpu
from jax.experimental.pallas import tpu_sc as plsc

_E = 1600000
_NGE = _E // 128
_NODES = 50000
_CELLS = 100000
_CELLSP = 100096
_NGC = _CELLSP // 128
_NC, _NS = 2, 16
_NW = _NC * _NS
_CH = 8
_EC = _CH * 128
_NCHK = _NGE // _CH
_NPS = _NODES // _NS
_ZR = 625


def _scatter_body(ea, nei, out0, out1, rbufA, rbufB, ibuf, zbuf, table,
                  lsem0, lsem1, ssem):
    c = lax.axis_index("c")
    s = lax.axis_index("s")
    w = s * _NC + c

    def _zb(i, carry):
        zbuf[i] = jnp.zeros((16,), jnp.float32)
        return carry

    lax.fori_loop(0, _ZR, _zb, 0)
    for k in range(_NPS // _ZR):
        pltpu.sync_copy(zbuf, table.at[pl.ds(s * _NPS + k * _ZR, _ZR)])
    plsc.subcore_barrier()

    cs = w * _NCHK // _NW
    ce = (w + 1) * _NCHK // _NW

    def _loads(ci, p, sem, start):
        g0 = ci * _CH
        e0 = g0 * 128
        pairs = (
            (ea.at[pl.ds(e0, _EC), pl.ds(0, 16)], rbufA.at[p]),
            (ea.at[pl.ds(e0, _EC), pl.ds(16, 16)], rbufB.at[p]),
            (nei.at[0, pl.ds(g0, _CH)], ibuf.at[0, p]),
            (nei.at[1, pl.ds(g0, _CH)], ibuf.at[1, p]),
        )
        for src, dst in pairs:
            if start:
                pltpu.async_copy(src, dst, sem)
            else:
                pltpu.make_async_copy(src, dst, sem).wait()

    def _drain(q):
        for j in range(_CH):
            pltpu.make_async_copy(rbufA.at[q, pl.ds(j * 128, 128)],
                                  table.at[ibuf.at[0, q, j]], ssem).wait()
            pltpu.make_async_copy(rbufB.at[q, pl.ds(j * 128, 128)],
                                  table.at[ibuf.at[1, q, j]], ssem).wait()

    def _chunk(ci, carry):
        p = lax.rem(ci - cs, 2)

        @pl.when(p == 0)
        def _():
            _loads(ci, 0, lsem0, start=False)

        @pl.when(p == 1)
        def _():
            _loads(ci, 1, lsem1, start=False)

        @pl.when(ci - 1 >= cs)
        def _():
            _drain(1 - p)

        @pl.when((p == 0) & (ci + 1 < ce))
        def _():
            _loads(ci + 1, 1, lsem1, start=True)

        @pl.when((p == 1) & (ci + 1 < ce))
        def _():
            _loads(ci + 1, 0, lsem0, start=True)

        for j in range(_CH):
            pltpu.async_copy(rbufA.at[p, pl.ds(j * 128, 128)],
                             table.at[ibuf.at[0, p, j]], ssem, add=True)
            pltpu.async_copy(rbufB.at[p, pl.ds(j * 128, 128)],
                             table.at[ibuf.at[1, p, j]], ssem, add=True)
        return carry

    @pl.when(cs < ce)
    def _():
        _loads(cs, 0, lsem0, start=True)

    lax.fori_loop(cs, ce, _chunk, 0)

    @pl.when(cs < ce)
    def _():
        _drain(lax.rem(ce - 1 - cs, 2))

    @pl.when(w == 0)
    def _():
        e0 = _NCHK * _EC
        g0 = _NCHK * _CH
        pltpu.sync_copy(ea.at[pl.ds(e0, 512), pl.ds(0, 16)],
                        rbufA.at[0, pl.ds(0, 512)])
        pltpu.sync_copy(ea.at[pl.ds(e0, 512), pl.ds(16, 16)],
                        rbufB.at[0, pl.ds(0, 512)])
        pltpu.sync_copy(nei.at[0, pl.ds(g0, 4)], ibuf.at[0, 0, pl.ds(0, 4)])
        pltpu.sync_copy(nei.at[1, pl.ds(g0, 4)], ibuf.at[1, 0, pl.ds(0, 4)])
        for j in range(4):
            pltpu.sync_copy(rbufA.at[0, pl.ds(j * 128, 128)],
                            table.at[ibuf.at[0, 0, j]], add=True)
            pltpu.sync_copy(rbufB.at[0, pl.ds(j * 128, 128)],
                            table.at[ibuf.at[1, 0, j]], add=True)

    plsc.subcore_barrier()

    @pl.when(c == 0)
    def _():
        pltpu.sync_copy(table.at[pl.ds(s * _NPS, _NPS)],
                        out0.at[pl.ds(s * _NPS, _NPS)])

    @pl.when(c == 1)
    def _():
        pltpu.sync_copy(table.at[pl.ds(s * _NPS, _NPS)],
                        out1.at[pl.ds(s * _NPS, _NPS)])


def _gather_body(p0, p1, fc, cells, ibuf, rbuf, obuf, isem0, isem1, gsem,
                 osem0, osem1):
    c = lax.axis_index("c")
    s = lax.axis_index("s")
    w = s * _NC + c
    gs = w * _NGC // _NW
    ge = (w + 1) * _NGC // _NW

    def _idx_loads(g, p, sem, start):
        for j in range(3):
            if start:
                pltpu.async_copy(fc.at[j, g], ibuf.at[p, j], sem)
            else:
                pltpu.make_async_copy(fc.at[j, g], ibuf.at[p, j], sem).wait()

    def _grp(g, carry):
        p = lax.rem(g - gs, 2)

        @pl.when(p == 0)
        def _():
            _idx_loads(g, 0, isem0, start=False)
            @pl.when(g - 2 >= gs)
            def _():
                pltpu.make_async_copy(
                    obuf.at[0], cells.at[pl.ds((g - 2) * 128, 128)],
                    osem0).wait()

        @pl.when(p == 1)
        def _():
            _idx_loads(g, 1, isem1, start=False)

            @pl.when(g - 2 >= gs)
            def _():
                pltpu.make_async_copy(
                    obuf.at[1], cells.at[pl.ds((g - 2) * 128, 128)],
                    osem1).wait()

        @pl.when((p == 0) & (g + 1 < ge))
        def _():
            _idx_loads(g + 1, 1, isem1, start=True)

        @pl.when((p == 1) & (g + 1 < ge))
        def _():
            _idx_loads(g + 1, 0, isem0, start=True)

        cps = []
        for j in range(3):
            cps.append(pltpu.async_copy(p0.at[ibuf.at[p, j]], rbuf.at[j],
                                        gsem))
            cps.append(pltpu.async_copy(p1.at[ibuf.at[p, j]], rbuf.at[3 + j],
                                        gsem))
        for cp in cps:
            cp.wait()

        def _cell(i, cc):
            acc = ((rbuf[0, i] + rbuf[1, i]) + (rbuf[2, i] + rbuf[3, i])
                   + (rbuf[4, i] + rbuf[5, i]))
            obuf[p, i] = acc
            return cc

        lax.fori_loop(0, 128, _cell, 0)

        @pl.when(p == 0)
        def _():
            pltpu.async_copy(obuf.at[0], cells.at[pl.ds(g * 128, 128)], osem0)

        @pl.when(p == 1)
        def _():
            pltpu.async_copy(obuf.at[1], cells.at[pl.ds(g * 128, 128)], osem1)

        return carry

    @pl.when(gs < ge)
    def _():
        _idx_loads(gs, 0, isem0, start=True)

    lax.fori_loop(gs, ge, _grp, 0)

    @pl.when(gs < ge)
    def _():
        n = ge - gs
        pg = lax.rem(n - 1, 2)

        @pl.when(pg == 0)
        def _():
            pltpu.make_async_copy(obuf.at[0],
                                  cells.at[pl.ds((ge - 1) * 128, 128)],
                                  osem0).wait()

        @pl.when(pg == 1)
        def _():
            pltpu.make_async_copy(obuf.at[1],
                                  cells.at[pl.ds((ge - 1) * 128, 128)],
                                  osem1).wait()

    @pl.when(gs < ge - 1)
    def _():
        n = ge - gs
        pg = lax.rem(n - 2, 2)

        @pl.when(pg == 0)
        def _():
            pltpu.make_async_copy(obuf.at[0],
                                  cells.at[pl.ds((ge - 2) * 128, 128)],
                                  osem0).wait()

        @pl.when(pg == 1)
        def _():
            pltpu.make_async_copy(obuf.at[1],
                                  cells.at[pl.ds((ge - 2) * 128, 128)],
                                  osem1).wait()


def _mm_body(x_ref, cl_ref, wx_ref, wc_ref, b_ref, o_ref):
    o_ref[...] = (jnp.dot(x_ref[...], wx_ref[...],
                          preferred_element_type=jnp.float32)
                  + jnp.dot(cl_ref[...], wc_ref[...],
                            preferred_element_type=jnp.float32)
                  + b_ref[...])


def _make_sc_kernels():
    mesh = plsc.VectorSubcoreMesh(core_axis_name="c", subcore_axis_name="s",
                                  num_cores=_NC, num_subcores=_NS)
    params = pltpu.CompilerParams(use_tc_tiling_on_sc=False)
    scatter = pl.kernel(
        _scatter_body,
        compiler_params=params,
        out_type=(jax.ShapeDtypeStruct((_NODES, 16), jnp.float32),
                  jax.ShapeDtypeStruct((_NODES, 16), jnp.float32)),
        mesh=mesh,
        scratch_types=[
            pltpu.VMEM((2, _EC, 16), jnp.float32),
            pltpu.VMEM((2, _EC, 16), jnp.float32),
            pltpu.VMEM((2, 2, _CH, 128), jnp.int32),
            pltpu.VMEM((_ZR, 16), jnp.float32),
            pltpu.VMEM_SHARED((_NODES, 16), jnp.float32),
            pltpu.SemaphoreType.DMA,
            pltpu.SemaphoreType.DMA,
            pltpu.SemaphoreType.DMA,
        ],
    )
    gather = pl.kernel(
        _gather_body,
        compiler_params=params,
        out_type=jax.ShapeDtypeStruct((_CELLSP, 16), jnp.float32),
        mesh=mesh,
        scratch_types=[
            pltpu.VMEM((2, 3, 128), jnp.int32),
            pltpu.VMEM((6, 128, 16), jnp.float32),
            pltpu.VMEM((2, 128, 16), jnp.float32),
            pltpu.SemaphoreType.DMA,
            pltpu.SemaphoreType.DMA,
            pltpu.SemaphoreType.DMA,
            pltpu.SemaphoreType.DMA,
            pltpu.SemaphoreType.DMA,
        ],
    )
    return scatter, gather


def _matmul(x, cl, wx, wc, b2):
    blk = 2000
    return pl.pallas_call(
        _mm_body,
        grid=(_CELLS // blk,),
        in_specs=[
            pl.BlockSpec((blk, 128), lambda i: (i, 0)),
            pl.BlockSpec((blk, 16), lambda i: (i, 0)),
            pl.BlockSpec((128, 128), lambda i: (0, 0)),
            pl.BlockSpec((16, 128), lambda i: (0, 0)),
            pl.BlockSpec((1, 128), lambda i: (0, 0)),
        ],
        out_specs=pl.BlockSpec((blk, 128), lambda i: (i, 0)),
        out_shape=jax.ShapeDtypeStruct((_CELLS, 128), jnp.float32),
    )(x, cl, wx, wc, b2)


def kernel(x, edge_attr, edge_index, node_edge_index, face, W, b):
    nei = node_edge_index.astype(jnp.int32).reshape(2, _NGE, 128)
    fcp = jnp.pad(face.astype(jnp.int32),
                  ((0, 0), (0, _CELLSP - _CELLS))).reshape(3, _NGC, 128)
    scatter, gather = _make_sc_kernels()
    p0, p1 = scatter(edge_attr, nei)
    cells = gather(p0, p1, fcp)
    wx = W[:128]
    wc = W[128:] * (1.0 / 3.0)
    out = _matmul(x, cells, wx, wc, b.reshape(1, 128))
    return (out, edge_attr, edge_index)


# speedup vs baseline: 4.8648x; 1.0016x over previous
import functools

import jax
import jax.numpy as jnp
from jax import lax
from jax.experimental import pallas as pl
from jax.experimental.pallas import tpu as pltpu
from jax.experimental.pallas import tpu_sc as plsc

_E = 1600000
_NGE = _E // 128
_NODES = 50000
_CELLS = 100000
_CELLSP = 100096
_NGC = _CELLSP // 128
_NC, _NS = 2, 16
_NW = _NC * _NS
_CH = 8
_EC = _CH * 128
_NCHK = _NGE // _CH
_NPS = _NODES // _NS
_ZR = 625


def _scatter_body(ea, nei, out0, out1, rbufA, rbufB, ibuf, zbuf, table,
                  lsem0, lsem1, ssem):
    c = lax.axis_index("c")
    s = lax.axis_index("s")
    w = s * _NC + c

    def _zb(i, carry):
        zbuf[i] = jnp.zeros((16,), jnp.float32)
        return carry

    lax.fori_loop(0, _ZR, _zb, 0)
    for k in range(_NPS // _ZR):
        pltpu.sync_copy(zbuf, table.at[pl.ds(s * _NPS + k * _ZR, _ZR)])
    plsc.subcore_barrier()

    cs = w * _NCHK // _NW
    ce = (w + 1) * _NCHK // _NW

    def _loads(ci, p, sem, start):
        g0 = ci * _CH
        e0 = g0 * 128
        pairs = (
            (ea.at[pl.ds(e0, _EC), pl.ds(0, 16)], rbufA.at[p]),
            (ea.at[pl.ds(e0, _EC), pl.ds(16, 16)], rbufB.at[p]),
            (nei.at[0, pl.ds(g0, _CH)], ibuf.at[0, p]),
            (nei.at[1, pl.ds(g0, _CH)], ibuf.at[1, p]),
        )
        for src, dst in pairs:
            if start:
                pltpu.async_copy(src, dst, sem)
            else:
                pltpu.make_async_copy(src, dst, sem).wait()

    def _chunk(ci, carry):
        p = lax.rem(ci - cs, 2)

        @pl.when(p == 0)
        def _():
            _loads(ci, 0, lsem0, start=False)

        @pl.when(p == 1)
        def _():
            _loads(ci, 1, lsem1, start=False)

        @pl.when((p == 0) & (ci + 1 < ce))
        def _():
            _loads(ci + 1, 1, lsem1, start=True)

        @pl.when((p == 1) & (ci + 1 < ce))
        def _():
            _loads(ci + 1, 0, lsem0, start=True)

        cps = []
        for j in range(_CH):
            cps.append(pltpu.async_copy(rbufA.at[p, pl.ds(j * 128, 128)],
                                        table.at[ibuf.at[0, p, j]], ssem,
                                        add=True))
            cps.append(pltpu.async_copy(rbufB.at[p, pl.ds(j * 128, 128)],
                                        table.at[ibuf.at[1, p, j]], ssem,
                                        add=True))
        for cp in cps:
            cp.wait()
        return carry

    @pl.when(cs < ce)
    def _():
        _loads(cs, 0, lsem0, start=True)

    lax.fori_loop(cs, ce, _chunk, 0)

    @pl.when(w == 0)
    def _():
        e0 = _NCHK * _EC
        g0 = _NCHK * _CH
        pltpu.sync_copy(ea.at[pl.ds(e0, 512), pl.ds(0, 16)],
                        rbufA.at[0, pl.ds(0, 512)])
        pltpu.sync_copy(ea.at[pl.ds(e0, 512), pl.ds(16, 16)],
                        rbufB.at[0, pl.ds(0, 512)])
        pltpu.sync_copy(nei.at[0, pl.ds(g0, 4)], ibuf.at[0, 0, pl.ds(0, 4)])
        pltpu.sync_copy(nei.at[1, pl.ds(g0, 4)], ibuf.at[1, 0, pl.ds(0, 4)])
        for j in range(4):
            pltpu.sync_copy(rbufA.at[0, pl.ds(j * 128, 128)],
                            table.at[ibuf.at[0, 0, j]], add=True)
            pltpu.sync_copy(rbufB.at[0, pl.ds(j * 128, 128)],
                            table.at[ibuf.at[1, 0, j]], add=True)

    plsc.subcore_barrier()

    @pl.when(c == 0)
    def _():
        pltpu.sync_copy(table.at[pl.ds(s * _NPS, _NPS)],
                        out0.at[pl.ds(s * _NPS, _NPS)])

    @pl.when(c == 1)
    def _():
        pltpu.sync_copy(table.at[pl.ds(s * _NPS, _NPS)],
                        out1.at[pl.ds(s * _NPS, _NPS)])


def _gather_body(p0, p1, fc, cells, ibuf, rbuf, obuf, isem0, isem1, gsem,
                 osem0, osem1):
    c = lax.axis_index("c")
    s = lax.axis_index("s")
    w = s * _NC + c
    gs = w * _NGC // _NW
    ge = (w + 1) * _NGC // _NW

    def _idx_loads(g, p, sem, start):
        for j in range(3):
            if start:
                pltpu.async_copy(fc.at[j, g], ibuf.at[p, j], sem)
            else:
                pltpu.make_async_copy(fc.at[j, g], ibuf.at[p, j], sem).wait()

    def _grp(g, carry):
        p = lax.rem(g - gs, 2)

        @pl.when(p == 0)
        def _():
            _idx_loads(g, 0, isem0, start=False)
            @pl.when(g - 2 >= gs)
            def _():
                pltpu.make_async_copy(
                    obuf.at[0], cells.at[pl.ds((g - 2) * 128, 128)],
                    osem0).wait()

        @pl.when(p == 1)
        def _():
            _idx_loads(g, 1, isem1, start=False)

            @pl.when(g - 2 >= gs)
            def _():
                pltpu.make_async_copy(
                    obuf.at[1], cells.at[pl.ds((g - 2) * 128, 128)],
                    osem1).wait()

        @pl.when((p == 0) & (g + 1 < ge))
        def _():
            _idx_loads(g + 1, 1, isem1, start=True)

        @pl.when((p == 1) & (g + 1 < ge))
        def _():
            _idx_loads(g + 1, 0, isem0, start=True)

        cps = []
        for j in range(3):
            cps.append(pltpu.async_copy(p0.at[ibuf.at[p, j]], rbuf.at[j],
                                        gsem))
            cps.append(pltpu.async_copy(p1.at[ibuf.at[p, j]], rbuf.at[3 + j],
                                        gsem))
        for cp in cps:
            cp.wait()

        def _cell(i, cc):
            acc = ((rbuf[0, i] + rbuf[1, i]) + (rbuf[2, i] + rbuf[3, i])
                   + (rbuf[4, i] + rbuf[5, i]))
            obuf[p, i] = acc
            return cc

        lax.fori_loop(0, 128, _cell, 0)

        @pl.when(p == 0)
        def _():
            pltpu.async_copy(obuf.at[0], cells.at[pl.ds(g * 128, 128)], osem0)

        @pl.when(p == 1)
        def _():
            pltpu.async_copy(obuf.at[1], cells.at[pl.ds(g * 128, 128)], osem1)

        return carry

    @pl.when(gs < ge)
    def _():
        _idx_loads(gs, 0, isem0, start=True)

    lax.fori_loop(gs, ge, _grp, 0)

    @pl.when(gs < ge)
    def _():
        n = ge - gs
        pg = lax.rem(n - 1, 2)

        @pl.when(pg == 0)
        def _():
            pltpu.make_async_copy(obuf.at[0],
                                  cells.at[pl.ds((ge - 1) * 128, 128)],
                                  osem0).wait()

        @pl.when(pg == 1)
        def _():
            pltpu.make_async_copy(obuf.at[1],
                                  cells.at[pl.ds((ge - 1) * 128, 128)],
                                  osem1).wait()

    @pl.when(gs < ge - 1)
    def _():
        n = ge - gs
        pg = lax.rem(n - 2, 2)

        @pl.when(pg == 0)
        def _():
            pltpu.make_async_copy(obuf.at[0],
                                  cells.at[pl.ds((ge - 2) * 128, 128)],
                                  osem0).wait()

        @pl.when(pg == 1)
        def _():
            pltpu.make_async_copy(obuf.at[1],
                                  cells.at[pl.ds((ge - 2) * 128, 128)],
                                  osem1).wait()


def _mm_body(x_ref, cl_ref, wx_ref, wc_ref, b_ref, o_ref):
    o_ref[...] = (jnp.dot(x_ref[...], wx_ref[...],
                          preferred_element_type=jnp.float32)
                  + jnp.dot(cl_ref[...], wc_ref[...],
                            preferred_element_type=jnp.float32)
                  + b_ref[...])


def _make_sc_kernels():
    mesh = plsc.VectorSubcoreMesh(core_axis_name="c", subcore_axis_name="s",
                                  num_cores=_NC, num_subcores=_NS)
    params = pltpu.CompilerParams(use_tc_tiling_on_sc=False)
    scatter = pl.kernel(
        _scatter_body,
        compiler_params=params,
        out_type=(jax.ShapeDtypeStruct((_NODES, 16), jnp.float32),
                  jax.ShapeDtypeStruct((_NODES, 16), jnp.float32)),
        mesh=mesh,
        scratch_types=[
            pltpu.VMEM((2, _EC, 16), jnp.float32),
            pltpu.VMEM((2, _EC, 16), jnp.float32),
            pltpu.VMEM((2, 2, _CH, 128), jnp.int32),
            pltpu.VMEM((_ZR, 16), jnp.float32),
            pltpu.VMEM_SHARED((_NODES, 16), jnp.float32),
            pltpu.SemaphoreType.DMA,
            pltpu.SemaphoreType.DMA,
            pltpu.SemaphoreType.DMA,
        ],
    )
    gather = pl.kernel(
        _gather_body,
        compiler_params=params,
        out_type=jax.ShapeDtypeStruct((_CELLSP, 16), jnp.float32),
        mesh=mesh,
        scratch_types=[
            pltpu.VMEM((2, 3, 128), jnp.int32),
            pltpu.VMEM((6, 128, 16), jnp.float32),
            pltpu.VMEM((2, 128, 16), jnp.float32),
            pltpu.SemaphoreType.DMA,
            pltpu.SemaphoreType.DMA,
            pltpu.SemaphoreType.DMA,
            pltpu.SemaphoreType.DMA,
            pltpu.SemaphoreType.DMA,
        ],
    )
    return scatter, gather


def _matmul(x, cl, wx, wc, b2):
    blk = 2000
    return pl.pallas_call(
        _mm_body,
        grid=(_CELLS // blk,),
        in_specs=[
            pl.BlockSpec((blk, 128), lambda i: (i, 0)),
            pl.BlockSpec((blk, 16), lambda i: (i, 0)),
            pl.BlockSpec((128, 128), lambda i: (0, 0)),
            pl.BlockSpec((16, 128), lambda i: (0, 0)),
            pl.BlockSpec((1, 128), lambda i: (0, 0)),
        ],
        out_specs=pl.BlockSpec((blk, 128), lambda i: (i, 0)),
        out_shape=jax.ShapeDtypeStruct((_CELLS, 128), jnp.float32),
    )(x, cl, wx, wc, b2)


def kernel(x, edge_attr, edge_index, node_edge_index, face, W, b):
    nei = node_edge_index.astype(jnp.int32).reshape(2, _NGE, 128)
    fcp = jnp.pad(face.astype(jnp.int32),
                  ((0, 0), (0, _CELLSP - _CELLS))).reshape(3, _NGC, 128)
    scatter, gather = _make_sc_kernels()
    p0, p1 = scatter(edge_attr, nei)
    cells = gather(p0, p1, fcp)
    wx = W[:128]
    wc = W[128:] * (1.0 / 3.0)
    out = _matmul(x, cells, wx, wc, b.reshape(1, 128))
    return (out, edge_attr, edge_index)
